# baseline (speedup 1.0000x reference)
import functools

import jax
import jax.numpy as jnp
from jax import lax
from jax.experimental import pallas as pl
from jax.experimental.pallas import tpu as pltpu
from jax.experimental.pallas import tpu_sc as plsc

_B = 4096
_L = 200
_D = 512
_CH = 40
_NCH = _L // _CH
_NC = 2
_NS = 16
_NW = _NC * _NS
_BPW = _B // _NW
_LANES = 16
_NBUF = 4


def _body(idx_hbm, pos_hbm, table_hbm, out_hbm, idx_v, pos_v,
          buf0, buf1, buf2, buf3,
          gsem0, gsem1, gsem2, gsem3, wsem0, wsem1, wsem2, wsem3):
    c = lax.axis_index("c")
    s = lax.axis_index("s")
    wid = s * _NC + c
    base = wid * _BPW
    bufs = (buf0, buf1, buf2, buf3)
    gsems = (gsem0, gsem1, gsem2, gsem3)
    wsems = (wsem0, wsem1, wsem2, wsem3)

    def start_gather(bl, p):
        pltpu.async_copy(
            table_hbm.at[idx_v.at[pl.ds(bl * _CH, _CH)]], bufs[p], gsems[p])

    def wait_gather(bl, p):
        pltpu.make_async_copy(
            table_hbm.at[idx_v.at[pl.ds(bl * _CH, _CH)]], bufs[p],
            gsems[p]).wait()

    def out_slice(bl, ch):
        row0 = (base + bl) * _L + ch * _CH
        return out_hbm.at[pl.ds(row0, _CH)]

    def add(p):
        buf = bufs[p]

        @plsc.parallel_loop(0, _CH, unroll=2)
        def r_body(r):
            for jj in range(_D // _LANES):
                sl = pl.ds(jj * _LANES, _LANES)
                buf[r, sl] = buf[r, sl] + pos_v[r, sl]

    def start_write(bl, p, ch):
        pltpu.async_copy(bufs[p], out_slice(bl, ch), wsems[p])

    def wait_write(bl, p, ch):
        pltpu.make_async_copy(bufs[p], out_slice(bl, ch), wsems[p]).wait()

    def ch_body(ch, _):
        pltpu.sync_copy(
            idx_hbm.at[pl.ds(ch * _B * _CH + base * _CH, _BPW * _CH)], idx_v)
        pltpu.sync_copy(pos_hbm.at[pl.ds(ch * _CH, _CH)], pos_v)

        start_gather(0, 0)
        start_gather(1, 1)
        start_gather(2, 2)

        wait_gather(0, 0)
        start_gather(3, 3)
        add(0)
        start_write(0, 0, ch)

        def quad_body(i, _):
            for k in range(_NBUF):
                bl = _NBUF * i + 1 + k
                p = (1 + k) % _NBUF
                q = (p + 3) % _NBUF
                wait_gather(bl, p)
                wait_write(bl - 1, q, ch)
                start_gather(bl + 3, q)
                add(p)
                start_write(bl, p, ch)
            return 0

        lax.fori_loop(0, (_BPW - _NBUF) // _NBUF, quad_body, 0)

        for bl in range(_BPW - 3, _BPW):
            p = bl % _NBUF
            wait_gather(bl, p)
            add(p)
            start_write(bl, p, ch)

        for bl in range(_BPW - _NBUF, _BPW):
            wait_write(bl, bl % _NBUF, ch)
        return 0

    lax.fori_loop(0, _NCH, ch_body, 0)


@jax.jit
def kernel(inputs, table, pos_embed):
    idx_r = (inputs.astype(jnp.int32)
             .reshape(_B, _NCH, _CH)
             .transpose(1, 0, 2)
             .reshape(_NCH * _B * _CH))
    pos2 = pos_embed.reshape(_L, _D)
    mesh = plsc.VectorSubcoreMesh(core_axis_name="c", subcore_axis_name="s")
    run = pl.kernel(
        _body,
        out_type=jax.ShapeDtypeStruct((_B * _L, _D), jnp.float32),
        mesh=mesh,
        scratch_types=(
            [pltpu.VMEM((_BPW * _CH,), jnp.int32),
             pltpu.VMEM((_CH, _D), jnp.float32)]
            + [pltpu.VMEM((_CH, _D), jnp.float32) for _ in range(_NBUF)]
            + [pltpu.SemaphoreType.DMA] * (2 * _NBUF)
        ),
    )
    out = run(idx_r, pos2, table)
    return out.reshape(_B, _L, _D)
